# baseline (speedup 1.0000x reference)
import functools

import jax
import jax.numpy as jnp
from jax import lax
from jax.experimental import pallas as pl
from jax.experimental.pallas import tpu as pltpu
from jax.experimental.pallas import tpu_sc as plsc

D_MODEL = 64
IDX_ROW = 128
SUB = 4
CHUNK = SUB * IDX_ROW


def _embed_sc(x_rows, table, n_rows):
    info = plsc.get_sparse_core_info()
    nc, ns = info.num_cores, info.num_subcores
    nw = nc * ns
    rows_per_w = n_rows // nw
    n_sup = rows_per_w // SUB
    b = n_rows * IDX_ROW

    mesh = plsc.VectorSubcoreMesh(core_axis_name="c", subcore_axis_name="s")

    @functools.partial(
        pl.kernel,
        mesh=mesh,
        out_type=jax.ShapeDtypeStruct((b, D_MODEL), jnp.float32),
        scratch_types=[
            pltpu.VMEM((SUB, IDX_ROW), jnp.int32),
            pltpu.VMEM((CHUNK, D_MODEL), jnp.float32),
            pltpu.SemaphoreType.DMA,
        ],
        compiler_params=pltpu.CompilerParams(use_tc_tiling_on_sc=False),
    )
    def k(x_hbm, table_hbm, out_hbm, idx_v, rows_v, sem):
        wid = lax.axis_index("s") * nc + lax.axis_index("c")
        wrow = wid * rows_per_w

        def chunk(i, carry):
            r0 = wrow + i * SUB
            pltpu.sync_copy(x_hbm.at[pl.ds(r0, SUB)], idx_v)
            cps = [
                pltpu.async_copy(
                    table_hbm.at[idx_v.at[j]],
                    rows_v.at[pl.ds(j * IDX_ROW, IDX_ROW)],
                    sem,
                )
                for j in range(SUB)
            ]
            for c in cps:
                c.wait()
            pltpu.sync_copy(rows_v, out_hbm.at[pl.ds(r0 * IDX_ROW, CHUNK)])
            return carry

        lax.fori_loop(0, n_sup, chunk, 0)

    return k(x_rows, table)


def kernel(x, table):
    orig_shape = x.shape
    xf = x.reshape(-1, IDX_ROW).astype(jnp.int32)
    out = _embed_sc(xf, table, xf.shape[0])
    return out.reshape(*orig_shape, D_MODEL)


# speedup vs baseline: 1.0456x; 1.0456x over previous
import functools

import jax
import jax.numpy as jnp
from jax import lax
from jax.experimental import pallas as pl
from jax.experimental.pallas import tpu as pltpu
from jax.experimental.pallas import tpu_sc as plsc

D_MODEL = 64
IDX_ROW = 128
SUB = 4
CHUNK = SUB * IDX_ROW
NBUF = 2


def _embed_sc(x_rows, table, n_rows):
    info = plsc.get_sparse_core_info()
    nc, ns = info.num_cores, info.num_subcores
    nw = nc * ns
    rows_per_w = n_rows // nw
    n_sup = rows_per_w // SUB
    b = n_rows * IDX_ROW

    mesh = plsc.VectorSubcoreMesh(core_axis_name="c", subcore_axis_name="s")

    @functools.partial(
        pl.kernel,
        mesh=mesh,
        out_type=jax.ShapeDtypeStruct((b, D_MODEL), jnp.float32),
        scratch_types=[
            pltpu.VMEM((NBUF, SUB, IDX_ROW), jnp.int32),
            pltpu.VMEM((NBUF, CHUNK, D_MODEL), jnp.float32),
            pltpu.SemaphoreType.DMA,
            pltpu.SemaphoreType.DMA,
            pltpu.SemaphoreType.DMA,
        ],
        compiler_params=pltpu.CompilerParams(use_tc_tiling_on_sc=False),
    )
    def k(x_hbm, table_hbm, out_hbm, idx_v, rows_v, isem, gsem, osem):
        wid = lax.axis_index("s") * nc + lax.axis_index("c")
        wrow = wid * rows_per_w

        def drain_store():
            pltpu.make_async_copy(
                rows_v.at[0], out_hbm.at[pl.ds(0, CHUNK)], osem
            ).wait()

        for j in range(min(NBUF, n_sup)):
            pltpu.async_copy(
                x_hbm.at[pl.ds(wrow + j * SUB, SUB)], idx_v.at[j], isem
            )

        def chunk(g, carry):
            buf = g % NBUF

            @pl.when(g >= NBUF)
            def _():
                drain_store()

            pltpu.make_async_copy(
                x_hbm.at[pl.ds(0, SUB)], idx_v.at[0], isem
            ).wait()

            cps = [
                pltpu.async_copy(
                    table_hbm.at[idx_v.at[buf].at[j]],
                    rows_v.at[buf].at[pl.ds(j * IDX_ROW, IDX_ROW)],
                    gsem,
                )
                for j in range(SUB)
            ]
            for c in cps:
                c.wait()

            @pl.when(g + NBUF < n_sup)
            def _():
                pltpu.async_copy(
                    x_hbm.at[pl.ds(wrow + (g + NBUF) * SUB, SUB)],
                    idx_v.at[buf],
                    isem,
                )

            pltpu.async_copy(
                rows_v.at[buf],
                out_hbm.at[pl.ds((wrow + g * SUB) * IDX_ROW, CHUNK)],
                osem,
            )
            return carry

        lax.fori_loop(0, n_sup, chunk, 0)

        for _ in range(min(NBUF, n_sup)):
            drain_store()

    return k(x_rows, table)


def kernel(x, table):
    orig_shape = x.shape
    xf = x.reshape(-1, IDX_ROW).astype(jnp.int32)
    out = _embed_sc(xf, table, xf.shape[0])
    return out.reshape(*orig_shape, D_MODEL)
